# baseline (speedup 1.0000x reference)
import functools

import jax
import jax.numpy as jnp
from jax import lax
from jax.experimental import pallas as pl
from jax.experimental.pallas import tpu as pltpu
from jax.experimental.pallas import tpu_sc as plsc

N = 100000
E = 3200000
F = 4
HID = 64
NC, NS = 2, 16
E_PER_W = E // (NC * NS)
CB = 2000
KB = E_PER_W // CB
SLICE_R = 6256
BLK = 8192
GRID = 13
NP = BLK * GRID

_MESH = plsc.VectorSubcoreMesh(
    core_axis_name="c", subcore_axis_name="s", num_cores=NC, num_subcores=NS)
_PARAMS = pltpu.CompilerParams(
    needs_layout_passes=False, use_tc_tiling_on_sc=False)

_AGG_SCRATCH = [
    pltpu.VMEM_SHARED((N,), jnp.float32),
    pltpu.VMEM((N,), jnp.float32),
    [pltpu.VMEM((CB,), jnp.int32) for _ in range(2)],
    [pltpu.VMEM((CB,), jnp.int32) for _ in range(2)],
    [pltpu.VMEM((CB,), jnp.float32) for _ in range(2)],
    [pltpu.VMEM((CB,), jnp.int32) for _ in range(2)],
    [pltpu.VMEM((CB,), jnp.float32) for _ in range(2)],
    pltpu.SemaphoreType.DMA,
    pltpu.SemaphoreType.DMA,
]


def _zero_acc_slice(acc, m0, r0):
    z16 = jnp.zeros((16,), jnp.float32)

    def zb(i, carry):
        m0[pl.ds(i * 16, 16)] = z16
        return carry
    lax.fori_loop(0, CB // 16, zb, 0)
    nz = SLICE_R // CB + 1

    def zcp(i, carry):
        o2 = jnp.minimum(r0 + i * CB, r0 + SLICE_R - CB)
        pltpu.sync_copy(m0, acc.at[pl.ds(o2, CB)])
        return carry
    lax.fori_loop(0, nz, zcp, 0)


def _edge_round(ei_hbm, w_hbm, tab, acc, base,
                src_v, dst_v, w_v, dsts_v, m_v, sem_ld, sem_s):
    def issue_loads(k, b):
        off = base + k * CB
        pltpu.async_copy(ei_hbm.at[0, pl.ds(off, CB)], src_v[b], sem_ld)
        pltpu.async_copy(ei_hbm.at[1, pl.ds(off, CB)], dst_v[b], sem_ld)
        pltpu.async_copy(w_hbm.at[pl.ds(off, CB)], w_v[b], sem_ld)

    def wait_loads(b):
        pltpu.make_async_copy(ei_hbm.at[0, pl.ds(0, CB)], src_v[b],
                              sem_ld).wait()
        pltpu.make_async_copy(ei_hbm.at[1, pl.ds(0, CB)], dst_v[b],
                              sem_ld).wait()
        pltpu.make_async_copy(w_hbm.at[pl.ds(0, CB)], w_v[b], sem_ld).wait()

    def wait_scatter(q):
        pltpu.make_async_copy(m_v[q], acc.at[dsts_v[q]], sem_s).wait()

    issue_loads(0, 0)

    def pair(k2, carry):
        for b in range(2):
            k = k2 * 2 + b

            @pl.when(k < KB - 1)
            def _():
                issue_loads(k + 1, 1 - b)
            wait_loads(b)

            @pl.when(k >= 2)
            def _():
                wait_scatter(b)

            def scale(i, carry2):
                sl = pl.ds(i * 16, 16)
                gv = plsc.load_gather(tab, [src_v[b][sl]])
                m_v[b][sl] = gv * w_v[b][sl]
                dsts_v[b][sl] = dst_v[b][sl]
                return carry2
            lax.fori_loop(0, CB // 16, scale, 0, unroll=8)
            pltpu.async_copy(m_v[b], acc.at[dsts_v[b]], sem_s, add=True)
        return carry
    lax.fori_loop(0, KB // 2, pair, 0)
    wait_scatter(0)
    wait_scatter(1)


@functools.partial(
    pl.kernel,
    out_type=jax.ShapeDtypeStruct((NC * F, N), jnp.float32),
    mesh=_MESH,
    scratch_types=_AGG_SCRATCH,
    compiler_params=_PARAMS,
)
def _agg4(ei_hbm, w_hbm, x0, x1, x2, x3, out_hbm,
          acc, xtab, src_v, dst_v, w_v, dsts_v, m_v, sem_ld, sem_s):
    c = lax.axis_index("c")
    s = lax.axis_index("s")
    wid = c * NS + s
    r0 = jnp.minimum(s * SLICE_R, N - SLICE_R)
    base = wid * E_PER_W

    for f, xf in enumerate((x0, x1, x2, x3)):
        gd = pltpu.async_copy(xf, xtab, sem_ld)
        _zero_acc_slice(acc, m_v[0], r0)
        gd.wait()
        plsc.subcore_barrier()
        _edge_round(ei_hbm, w_hbm, xtab, acc, base,
                    src_v, dst_v, w_v, dsts_v, m_v, sem_ld, sem_s)
        plsc.subcore_barrier()
        pltpu.sync_copy(acc.at[pl.ds(r0, SLICE_R)],
                        out_hbm.at[c * F + f, pl.ds(r0, SLICE_R)])
        plsc.subcore_barrier()


@functools.partial(
    pl.kernel,
    out_type=jax.ShapeDtypeStruct((NC, N), jnp.float32),
    mesh=_MESH,
    scratch_types=_AGG_SCRATCH,
    compiler_params=_PARAMS,
)
def _agg1(ei_hbm, w_hbm, g_hbm, out_hbm,
          acc, gtab, src_v, dst_v, w_v, dsts_v, m_v, sem_ld, sem_s):
    c = lax.axis_index("c")
    s = lax.axis_index("s")
    wid = c * NS + s
    r0 = jnp.minimum(s * SLICE_R, N - SLICE_R)
    base = wid * E_PER_W

    gd = pltpu.async_copy(g_hbm.at[pl.ds(0, N)], gtab, sem_ld)
    _zero_acc_slice(acc, m_v[0], r0)
    gd.wait()
    plsc.subcore_barrier()
    _edge_round(ei_hbm, w_hbm, gtab, acc, base,
                src_v, dst_v, w_v, dsts_v, m_v, sem_ld, sem_s)
    plsc.subcore_barrier()
    pltpu.sync_copy(acc.at[pl.ds(r0, SLICE_R)],
                    out_hbm.at[c, pl.ds(r0, SLICE_R)])


def _mid_body(p, x0, x1, x2, x3, w1, b1, wr1, w2, wr2, b2, g_out, rb_out):
    pv = p[...]
    aggT = pv[0:F] + pv[F:2 * F]
    w1v, wr1v = w1[...], wr1[...]
    preT = jnp.broadcast_to(b1[...][:, None], (HID, BLK))
    for f, xf in enumerate((x0, x1, x2, x3)):
        preT = preT + w1v[f][:, None] * aggT[f][None, :]
        preT = preT + wr1v[f][:, None] * xf[...][None, :]
    hT = jnp.maximum(preT, 0.0)
    g_out[...] = jnp.sum(hT * w2[...], axis=0)
    rb_out[...] = jnp.sum(hT * wr2[...], axis=0) + b2[...][0]


def _fin_body(sp, rb, out):
    spv = sp[...]
    out[...] = (spv[0] + spv[1] + rb[...])[:, None]


def _full_spec(shape):
    nd = len(shape)
    return pl.BlockSpec(shape, lambda i: (0,) * nd)


def _vec_spec():
    return pl.BlockSpec((BLK,), lambda i: (i,))


_mid = pl.pallas_call(
    _mid_body,
    grid=(GRID,),
    in_specs=[
        pl.BlockSpec((NC * F, BLK), lambda i: (0, i)),
        _vec_spec(), _vec_spec(), _vec_spec(), _vec_spec(),
        _full_spec((F, HID)), _full_spec((HID,)), _full_spec((F, HID)),
        _full_spec((HID, 1)), _full_spec((HID, 1)), _full_spec((1,)),
    ],
    out_specs=[_vec_spec(), _vec_spec()],
    out_shape=[jax.ShapeDtypeStruct((NP,), jnp.float32),
               jax.ShapeDtypeStruct((NP,), jnp.float32)],
)

_fin = pl.pallas_call(
    _fin_body,
    grid=(GRID,),
    in_specs=[pl.BlockSpec((NC, BLK), lambda i: (0, i)),
              _vec_spec()],
    out_specs=pl.BlockSpec((BLK, 1), lambda i: (i, 0)),
    out_shape=jax.ShapeDtypeStruct((N, 1), jnp.float32),
)


def kernel(x, edge_index, edge_attr, W_rel1, b_rel1, W_root1,
           W_rel2, b_rel2, W_root2):
    ei = edge_index.astype(jnp.int32)
    w = edge_attr.astype(jnp.float32)
    xcols = [x[:, f] for f in range(F)]

    p = _agg4(ei, w, *xcols)
    g, rb = _mid(p, *xcols, W_rel1, b_rel1, W_root1,
                 W_rel2, W_root2, b_rel2)
    sp = _agg1(ei, w, g)
    return _fin(sp, rb)


# speedup vs baseline: 1.3955x; 1.3955x over previous
import functools

import jax
import jax.numpy as jnp
from jax import lax
from jax.experimental import pallas as pl
from jax.experimental.pallas import tpu as pltpu
from jax.experimental.pallas import tpu_sc as plsc

N = 100000
E = 3200000
F = 4
HID = 64
NC, NS = 2, 16
E_PER_W = E // (NC * NS)
CB = 2000
KB = E_PER_W // CB
SLICE_R = 6256
BLK = 8192
GRID = 13
NP = BLK * GRID

_MESH = plsc.VectorSubcoreMesh(
    core_axis_name="c", subcore_axis_name="s", num_cores=NC, num_subcores=NS)
_PARAMS = pltpu.CompilerParams(
    needs_layout_passes=False, use_tc_tiling_on_sc=False)

_AGG_SCRATCH = [
    pltpu.VMEM_SHARED((N,), jnp.float32),
    pltpu.VMEM((N,), jnp.float32),
    [pltpu.VMEM((CB,), jnp.int32) for _ in range(2)],
    [pltpu.VMEM((CB,), jnp.int32) for _ in range(2)],
    [pltpu.VMEM((CB,), jnp.float32) for _ in range(2)],
    [pltpu.VMEM((CB,), jnp.int32) for _ in range(2)],
    [pltpu.VMEM((CB,), jnp.float32) for _ in range(2)],
    pltpu.SemaphoreType.DMA,
    pltpu.SemaphoreType.DMA,
]


def _zero_acc_slice(acc, m0, r0):
    z16 = jnp.zeros((16,), jnp.float32)

    def zb(i, carry):
        m0[pl.ds(i * 16, 16)] = z16
        return carry
    lax.fori_loop(0, CB // 16, zb, 0)
    nz = SLICE_R // CB + 1

    def zcp(i, carry):
        o2 = jnp.minimum(r0 + i * CB, r0 + SLICE_R - CB)
        pltpu.sync_copy(m0, acc.at[pl.ds(o2, CB)])
        return carry
    lax.fori_loop(0, nz, zcp, 0)


def _edge_round(ei_hbm, w_hbm, tab, acc, base,
                src_v, dst_v, w_v, dsts_v, m_v, sem_ld, sem_s):
    def issue_loads(k, b):
        off = base + k * CB
        pltpu.async_copy(ei_hbm.at[0, pl.ds(off, CB)], src_v[b], sem_ld)
        pltpu.async_copy(ei_hbm.at[1, pl.ds(off, CB)], dst_v[b], sem_ld)
        pltpu.async_copy(w_hbm.at[pl.ds(off, CB)], w_v[b], sem_ld)

    def wait_loads(b):
        pltpu.make_async_copy(ei_hbm.at[0, pl.ds(0, CB)], src_v[b],
                              sem_ld).wait()
        pltpu.make_async_copy(ei_hbm.at[1, pl.ds(0, CB)], dst_v[b],
                              sem_ld).wait()
        pltpu.make_async_copy(w_hbm.at[pl.ds(0, CB)], w_v[b], sem_ld).wait()

    def wait_scatter(q):
        pltpu.make_async_copy(m_v[q], acc.at[dsts_v[q]], sem_s).wait()

    issue_loads(0, 0)

    def pair(k2, carry):
        for b in range(2):
            k = k2 * 2 + b

            @pl.when(k < KB - 1)
            def _():
                issue_loads(k + 1, 1 - b)
            wait_loads(b)

            @pl.when(k >= 2)
            def _():
                wait_scatter(b)

            def scale(i, carry2):
                sl = pl.ds(i * 16, 16)
                gv = plsc.load_gather(tab, [src_v[b][sl]])
                m_v[b][sl] = gv * w_v[b][sl]
                dsts_v[b][sl] = dst_v[b][sl]
                return carry2
            lax.fori_loop(0, CB // 16, scale, 0)
            pltpu.async_copy(m_v[b], acc.at[dsts_v[b]], sem_s, add=True)
        return carry
    lax.fori_loop(0, KB // 2, pair, 0)
    wait_scatter(0)
    wait_scatter(1)


@functools.partial(
    pl.kernel,
    out_type=jax.ShapeDtypeStruct((NC * F, N), jnp.float32),
    mesh=_MESH,
    scratch_types=_AGG_SCRATCH,
    compiler_params=_PARAMS,
)
def _agg4(ei_hbm, w_hbm, x0, x1, x2, x3, out_hbm,
          acc, xtab, src_v, dst_v, w_v, dsts_v, m_v, sem_ld, sem_s):
    c = lax.axis_index("c")
    s = lax.axis_index("s")
    wid = c * NS + s
    r0 = jnp.minimum(s * SLICE_R, N - SLICE_R)
    base = wid * E_PER_W

    for f, xf in enumerate((x0, x1, x2, x3)):
        gd = pltpu.async_copy(xf, xtab, sem_ld)
        _zero_acc_slice(acc, m_v[0], r0)
        gd.wait()
        plsc.subcore_barrier()
        _edge_round(ei_hbm, w_hbm, xtab, acc, base,
                    src_v, dst_v, w_v, dsts_v, m_v, sem_ld, sem_s)
        plsc.subcore_barrier()
        pltpu.sync_copy(acc.at[pl.ds(r0, SLICE_R)],
                        out_hbm.at[c * F + f, pl.ds(r0, SLICE_R)])
        plsc.subcore_barrier()


@functools.partial(
    pl.kernel,
    out_type=jax.ShapeDtypeStruct((NC, N), jnp.float32),
    mesh=_MESH,
    scratch_types=_AGG_SCRATCH,
    compiler_params=_PARAMS,
)
def _agg1(ei_hbm, w_hbm, g_hbm, out_hbm,
          acc, gtab, src_v, dst_v, w_v, dsts_v, m_v, sem_ld, sem_s):
    c = lax.axis_index("c")
    s = lax.axis_index("s")
    wid = c * NS + s
    r0 = jnp.minimum(s * SLICE_R, N - SLICE_R)
    base = wid * E_PER_W

    gd = pltpu.async_copy(g_hbm.at[pl.ds(0, N)], gtab, sem_ld)
    _zero_acc_slice(acc, m_v[0], r0)
    gd.wait()
    plsc.subcore_barrier()
    _edge_round(ei_hbm, w_hbm, gtab, acc, base,
                src_v, dst_v, w_v, dsts_v, m_v, sem_ld, sem_s)
    plsc.subcore_barrier()
    pltpu.sync_copy(acc.at[pl.ds(r0, SLICE_R)],
                    out_hbm.at[c, pl.ds(r0, SLICE_R)])


def _mid_body(p, x0, x1, x2, x3, w1, b1, wr1, w2, wr2, b2, g_out, rb_out):
    pv = p[...]
    aggT = pv[0:F] + pv[F:2 * F]
    w1v, wr1v = w1[...], wr1[...]
    preT = jnp.broadcast_to(b1[...][:, None], (HID, BLK))
    for f, xf in enumerate((x0, x1, x2, x3)):
        preT = preT + w1v[f][:, None] * aggT[f][None, :]
        preT = preT + wr1v[f][:, None] * xf[...][None, :]
    hT = jnp.maximum(preT, 0.0)
    g_out[...] = jnp.sum(hT * w2[...], axis=0)
    rb_out[...] = jnp.sum(hT * wr2[...], axis=0) + b2[...][0]


def _fin_body(sp, rb, out):
    spv = sp[...]
    out[...] = (spv[0] + spv[1] + rb[...])[:, None]


def _full_spec(shape):
    nd = len(shape)
    return pl.BlockSpec(shape, lambda i: (0,) * nd)


def _vec_spec():
    return pl.BlockSpec((BLK,), lambda i: (i,))


_mid = pl.pallas_call(
    _mid_body,
    grid=(GRID,),
    in_specs=[
        pl.BlockSpec((NC * F, BLK), lambda i: (0, i)),
        _vec_spec(), _vec_spec(), _vec_spec(), _vec_spec(),
        _full_spec((F, HID)), _full_spec((HID,)), _full_spec((F, HID)),
        _full_spec((HID, 1)), _full_spec((HID, 1)), _full_spec((1,)),
    ],
    out_specs=[_vec_spec(), _vec_spec()],
    out_shape=[jax.ShapeDtypeStruct((NP,), jnp.float32),
               jax.ShapeDtypeStruct((NP,), jnp.float32)],
)

_fin = pl.pallas_call(
    _fin_body,
    grid=(GRID,),
    in_specs=[pl.BlockSpec((NC, BLK), lambda i: (0, i)),
              _vec_spec()],
    out_specs=pl.BlockSpec((BLK, 1), lambda i: (i, 0)),
    out_shape=jax.ShapeDtypeStruct((N, 1), jnp.float32),
)


def kernel(x, edge_index, edge_attr, W_rel1, b_rel1, W_root1,
           W_rel2, b_rel2, W_root2):
    ei = edge_index.astype(jnp.int32)
    w = edge_attr.astype(jnp.float32)
    xcols = [x[:, f] for f in range(F)]

    p = _agg4(ei, w, *xcols)
    g, rb = _mid(p, *xcols, W_rel1, b_rel1, W_root1,
                 W_rel2, W_root2, b_rel2)
    sp = _agg1(ei, w, g)
    return _fin(sp, rb)


# speedup vs baseline: 1.6264x; 1.1654x over previous
import functools

import jax
import jax.numpy as jnp
from jax import lax
from jax.experimental import pallas as pl
from jax.experimental.pallas import tpu as pltpu
from jax.experimental.pallas import tpu_sc as plsc

N = 100000
E = 3200000
F = 4
HID = 64
NC, NS = 2, 16
E_PER_W = E // (NC * NS)
CB = 2000
KB = E_PER_W // CB
SLICE_R = 6256
BLK = 8192
GRID = 13
NP = BLK * GRID

_MESH = plsc.VectorSubcoreMesh(
    core_axis_name="c", subcore_axis_name="s", num_cores=NC, num_subcores=NS)
_PARAMS = pltpu.CompilerParams(
    needs_layout_passes=False, use_tc_tiling_on_sc=False)

_AGG_SCRATCH = [
    pltpu.VMEM_SHARED((N,), jnp.float32),
    pltpu.VMEM((N,), jnp.float32),
    [pltpu.VMEM((CB,), jnp.int32) for _ in range(2)],
    [pltpu.VMEM((CB,), jnp.int32) for _ in range(2)],
    [pltpu.VMEM((CB,), jnp.float32) for _ in range(2)],
    [pltpu.VMEM((CB,), jnp.int32) for _ in range(2)],
    [pltpu.VMEM((CB,), jnp.float32) for _ in range(2)],
    pltpu.SemaphoreType.DMA,
    pltpu.SemaphoreType.DMA,
]


def _zero_acc_slice(acc, m0, r0):
    z16 = jnp.zeros((16,), jnp.float32)

    def zb(i, carry):
        m0[pl.ds(i * 16, 16)] = z16
        return carry
    lax.fori_loop(0, CB // 16, zb, 0)
    nz = SLICE_R // CB + 1

    def zcp(i, carry):
        o2 = jnp.minimum(r0 + i * CB, r0 + SLICE_R - CB)
        pltpu.sync_copy(m0, acc.at[pl.ds(o2, CB)])
        return carry
    lax.fori_loop(0, nz, zcp, 0)


def _edge_round(ei_hbm, w_hbm, tab, acc, base,
                src_v, dst_v, w_v, dsts_v, m_v, sem_ld, sem_s):
    def issue_loads(k, b):
        off = base + k * CB
        pltpu.async_copy(ei_hbm.at[0, pl.ds(off, CB)], src_v[b], sem_ld)
        pltpu.async_copy(ei_hbm.at[1, pl.ds(off, CB)], dst_v[b], sem_ld)
        pltpu.async_copy(w_hbm.at[pl.ds(off, CB)], w_v[b], sem_ld)

    def wait_loads(b):
        pltpu.make_async_copy(ei_hbm.at[0, pl.ds(0, CB)], src_v[b],
                              sem_ld).wait()
        pltpu.make_async_copy(ei_hbm.at[1, pl.ds(0, CB)], dst_v[b],
                              sem_ld).wait()
        pltpu.make_async_copy(w_hbm.at[pl.ds(0, CB)], w_v[b], sem_ld).wait()

    def wait_scatter(q):
        pltpu.make_async_copy(m_v[q], acc.at[dsts_v[q]], sem_s).wait()

    issue_loads(0, 0)

    def pair(k2, carry):
        for b in range(2):
            k = k2 * 2 + b

            @pl.when(k < KB - 1)
            def _():
                issue_loads(k + 1, 1 - b)
            wait_loads(b)

            @pl.when(k >= 2)
            def _():
                wait_scatter(b)

            def scale(i, carry2):
                sl = pl.ds(i * 16, 16)
                gv = plsc.load_gather(tab, [src_v[b][sl]])
                m_v[b][sl] = gv * w_v[b][sl]
                dsts_v[b][sl] = dst_v[b][sl]
                return carry2
            lax.fori_loop(0, CB // 16, scale, 0)
            pltpu.async_copy(m_v[b], acc.at[dsts_v[b]], sem_s, add=True)
        return carry
    lax.fori_loop(0, KB // 2, pair, 0)
    wait_scatter(0)
    wait_scatter(1)


@functools.partial(
    pl.kernel,
    out_type=jax.ShapeDtypeStruct((NC * F, N), jnp.float32),
    mesh=_MESH,
    scratch_types=_AGG_SCRATCH,
    compiler_params=_PARAMS,
)
def _agg4(ei_hbm, w_hbm, x0, x1, x2, x3, out_hbm,
          acc, xtab, src_v, dst_v, w_v, dsts_v, m_v, sem_ld, sem_s):
    c = lax.axis_index("c")
    s = lax.axis_index("s")
    wid = c * NS + s
    r0 = jnp.minimum(s * SLICE_R, N - SLICE_R)
    base = wid * E_PER_W

    for f, xf in enumerate((x0, x1, x2, x3)):
        gd = pltpu.async_copy(xf, xtab, sem_ld)
        _zero_acc_slice(acc, m_v[0], r0)
        gd.wait()
        plsc.subcore_barrier()
        _edge_round(ei_hbm, w_hbm, xtab, acc, base,
                    src_v, dst_v, w_v, dsts_v, m_v, sem_ld, sem_s)
        plsc.subcore_barrier()
        pltpu.sync_copy(acc.at[pl.ds(r0, SLICE_R)],
                        out_hbm.at[c * F + f, pl.ds(r0, SLICE_R)])
        plsc.subcore_barrier()


@functools.partial(
    pl.kernel,
    out_type=jax.ShapeDtypeStruct((NC, N), jnp.float32),
    mesh=_MESH,
    scratch_types=_AGG_SCRATCH,
    compiler_params=_PARAMS,
)
def _agg1(ei_hbm, w_hbm, g_hbm, out_hbm,
          acc, gtab, src_v, dst_v, w_v, dsts_v, m_v, sem_ld, sem_s):
    c = lax.axis_index("c")
    s = lax.axis_index("s")
    wid = c * NS + s
    r0 = jnp.minimum(s * SLICE_R, N - SLICE_R)
    base = wid * E_PER_W

    gd = pltpu.async_copy(g_hbm.at[pl.ds(0, N)], gtab, sem_ld)
    _zero_acc_slice(acc, m_v[0], r0)
    gd.wait()
    plsc.subcore_barrier()
    _edge_round(ei_hbm, w_hbm, gtab, acc, base,
                src_v, dst_v, w_v, dsts_v, m_v, sem_ld, sem_s)
    plsc.subcore_barrier()
    pltpu.sync_copy(acc.at[pl.ds(r0, SLICE_R)],
                    out_hbm.at[c, pl.ds(r0, SLICE_R)])


def _mid_body(p, x0, x1, x2, x3, w1, b1, wr1, w2, wr2, b2, g_out, rb_out):
    pv = p[...]
    aggT = pv[0:F] + pv[F:2 * F]
    w1v, wr1v = w1[...], wr1[...]
    preT = jnp.broadcast_to(b1[...][:, None], (HID, BLK))
    for f, xf in enumerate((x0, x1, x2, x3)):
        preT = preT + w1v[f][:, None] * aggT[f][None, :]
        preT = preT + wr1v[f][:, None] * xf[...][None, :]
    hT = jnp.maximum(preT, 0.0)
    g_out[...] = jnp.dot(w2[...], hT,
                         preferred_element_type=jnp.float32)[0]
    rb_out[...] = (jnp.dot(wr2[...], hT,
                           preferred_element_type=jnp.float32)[0]
                   + b2[...][0])


def _fin_body(sp, rb, out):
    spv = sp[...]
    out[...] = spv[0] + spv[1] + rb[...]


def _full_spec(shape):
    nd = len(shape)
    return pl.BlockSpec(shape, lambda i: (0,) * nd)


def _vec_spec():
    return pl.BlockSpec((BLK,), lambda i: (i,))


_mid = pl.pallas_call(
    _mid_body,
    grid=(GRID,),
    in_specs=[
        pl.BlockSpec((NC * F, BLK), lambda i: (0, i)),
        _vec_spec(), _vec_spec(), _vec_spec(), _vec_spec(),
        _full_spec((F, HID)), _full_spec((HID,)), _full_spec((F, HID)),
        _full_spec((1, HID)), _full_spec((1, HID)), _full_spec((1,)),
    ],
    out_specs=[_vec_spec(), _vec_spec()],
    out_shape=[jax.ShapeDtypeStruct((NP,), jnp.float32),
               jax.ShapeDtypeStruct((NP,), jnp.float32)],
)

_fin = pl.pallas_call(
    _fin_body,
    grid=(GRID,),
    in_specs=[pl.BlockSpec((NC, BLK), lambda i: (0, i)),
              _vec_spec()],
    out_specs=_vec_spec(),
    out_shape=jax.ShapeDtypeStruct((NP,), jnp.float32),
)


def kernel(x, edge_index, edge_attr, W_rel1, b_rel1, W_root1,
           W_rel2, b_rel2, W_root2):
    ei = edge_index.astype(jnp.int32)
    w = edge_attr.astype(jnp.float32)
    xcols = [x[:, f] for f in range(F)]

    p = _agg4(ei, w, *xcols)
    g, rb = _mid(p, *xcols, W_rel1, b_rel1, W_root1,
                 W_rel2.reshape(1, HID), W_root2.reshape(1, HID),
                 b_rel2)
    sp = _agg1(ei, w, g)
    return _fin(sp, rb)[:N].reshape(N, 1)
